# Initial kernel scaffold; baseline (speedup 1.0000x reference)
import jax
import jax.numpy as jnp
from jax import lax
from jax.experimental import pallas as pl
from jax.experimental.pallas import tpu as pltpu
from jax.experimental.pallas import tpu_sc as plsc

N_POINTS = 819200
N_GRID = 512
N_BASIS = 16

NC = 2
NS = 16
NW = NC * NS

PER_W = N_POINTS // NW
CHUNK = 1024
N_CHUNKS = PER_W // CHUNK


def _sc_body(t_hbm, grid_hbm, out_hbm, table_v, t_v, out_v, sem):
    wid = lax.axis_index("s") * NC + lax.axis_index("c")
    my_base = wid * PER_W

    pltpu.sync_copy(grid_hbm, table_v)

    def chunk_body(ci, _):
        base = my_base + ci * CHUNK
        pltpu.sync_copy(t_hbm.at[pl.ds(base, CHUNK)], t_v)

        def point_body(i, _):
            tv = t_v[i]
            u = jnp.minimum(jnp.maximum(tv * jnp.float32(N_GRID - 1),
                                        jnp.float32(0.0)),
                            jnp.float32(N_GRID - 1))
            i0 = u.astype(jnp.int32)
            i1 = jnp.minimum(i0 + 1, N_GRID - 1)
            w = u - i0.astype(jnp.float32)
            row0 = table_v[i0, :]
            row1 = table_v[i1, :]
            out_v[i, :] = row0 + w * (row1 - row0)
            return 0

        lax.fori_loop(0, CHUNK, point_body, 0)
        pltpu.sync_copy(out_v, out_hbm.at[pl.ds(base, CHUNK), :])
        return 0

    lax.fori_loop(0, N_CHUNKS, chunk_body, 0)


def kernel(t, I_grid):
    mesh = plsc.VectorSubcoreMesh(core_axis_name="c", subcore_axis_name="s")
    f = pl.kernel(
        _sc_body,
        out_type=jax.ShapeDtypeStruct((N_POINTS, N_BASIS), jnp.float32),
        mesh=mesh,
        scratch_types=[
            pltpu.VMEM((N_GRID, N_BASIS), jnp.float32),
            pltpu.VMEM((CHUNK,), jnp.float32),
            pltpu.VMEM((CHUNK, N_BASIS), jnp.float32),
            pltpu.SemaphoreType.DMA,
        ],
    )
    return f(t, I_grid)


# speedup vs baseline: 8.7526x; 8.7526x over previous
import jax
import jax.numpy as jnp
from jax import lax
from jax.experimental import pallas as pl
from jax.experimental.pallas import tpu as pltpu
from jax.experimental.pallas import tpu_sc as plsc

N_POINTS = 819200
N_GRID = 512
N_BASIS = 16

NC = 2
NS = 16
NW = NC * NS

PER_W = N_POINTS // NW
CHUNK = 1024
N_CHUNKS = PER_W // CHUNK


def _sc_body(t_hbm, grid_hbm, out_hbm, table_v, t_v, out_v, sem):
    wid = lax.axis_index("s") * NC + lax.axis_index("c")
    my_base = wid * PER_W

    pltpu.sync_copy(grid_hbm, table_v)

    def chunk_body(ci, _):
        base = my_base + ci * CHUNK
        pltpu.sync_copy(t_hbm.at[pl.ds(base, CHUNK)], t_v)

        def block_body(j, _):
            tvec = t_v[pl.ds(j * 16, 16)]
            u = jnp.minimum(jnp.maximum(tvec * jnp.float32(N_GRID - 1),
                                        jnp.float32(0.0)),
                            jnp.float32(N_GRID - 1))
            i0v = u.astype(jnp.int32)
            wv = u - i0v.astype(jnp.float32)
            o0v = i0v * N_BASIS
            for k in range(16):
                o0 = o0v[k]
                o1 = jnp.minimum(o0 + N_BASIS, (N_GRID - 1) * N_BASIS)
                w = wv[k]
                row0 = table_v[pl.ds(o0, N_BASIS)]
                row1 = table_v[pl.ds(o1, N_BASIS)]
                out_v[pl.ds((j * 16 + k) * N_BASIS, N_BASIS)] = (
                    row0 + w * (row1 - row0))
            return 0

        lax.fori_loop(0, CHUNK // 16, block_body, 0)
        pltpu.sync_copy(out_v, out_hbm.at[pl.ds(base * N_BASIS,
                                                CHUNK * N_BASIS)])
        return 0

    lax.fori_loop(0, N_CHUNKS, chunk_body, 0)


def kernel(t, I_grid):
    mesh = plsc.VectorSubcoreMesh(core_axis_name="c", subcore_axis_name="s")
    f = pl.kernel(
        _sc_body,
        out_type=jax.ShapeDtypeStruct((N_POINTS * N_BASIS,), jnp.float32),
        mesh=mesh,
        scratch_types=[
            pltpu.VMEM((N_GRID * N_BASIS,), jnp.float32),
            pltpu.VMEM((CHUNK,), jnp.float32),
            pltpu.VMEM((CHUNK * N_BASIS,), jnp.float32),
            pltpu.SemaphoreType.DMA,
        ],
    )
    out_flat = f(t, I_grid.reshape(-1))
    return out_flat.reshape(N_POINTS, N_BASIS)
